# baseline (speedup 1.0000x reference)
import functools

import jax
import jax.numpy as jnp
from jax import lax
from jax.experimental import pallas as pl
from jax.experimental.pallas import tpu as pltpu
from jax.experimental.pallas import tpu_sc as plsc

_N = 100
_NCOL = 128
_A_FLAT = _N * _NCOL
_E = 6400
_NC, _NS, _L = 2, 16, 16
_ACT = 8
_NW = _NC * _ACT
_EPW = _E // _NW

_NT = (((1,), (1,)), ((), ()))


def _sc_count_body(src_hbm, dst_hbm, out_hbm, src_v, dst_v, hist):
    c = lax.axis_index("c")
    s = lax.axis_index("s")

    @pl.when(s < _ACT)
    def _():
        w = c * _ACT + s
        base = w * _EPW
        zeros = jnp.zeros((_L,), jnp.float32)
        for i in range(_A_FLAT // _L):
            hist[pl.ds(i * _L, _L)] = zeros
        pltpu.sync_copy(src_hbm.at[pl.ds(base, _EPW)], src_v)
        pltpu.sync_copy(dst_hbm.at[pl.ds(base, _EPW)], dst_v)
        lane = lax.iota(jnp.int32, _L)
        ones = jnp.ones((_L,), jnp.float32)
        for i in range(_EPW // _L):
            sv = src_v[pl.ds(i * _L, _L)]
            dv = dst_v[pl.ds(i * _L, _L)]
            off = dv * _NCOL + sv
            for l in range(_L):
                plsc.addupdate_scatter(hist, [off], ones, mask=lane == l)
        pltpu.sync_copy(hist, out_hbm.at[w])


_sc_count = functools.partial(
    pl.kernel,
    out_type=jax.ShapeDtypeStruct((_NW, _A_FLAT), jnp.float32),
    mesh=plsc.VectorSubcoreMesh(core_axis_name="c", subcore_axis_name="s"),
    compiler_params=pltpu.CompilerParams(needs_layout_passes=False),
    scratch_types=[
        pltpu.VMEM((_EPW,), jnp.int32),
        pltpu.VMEM((_EPW,), jnp.int32),
        pltpu.VMEM((_A_FLAT,), jnp.float32),
    ],
)(_sc_count_body)


def _tc_body(a2_ref, x_ref, w1_ref, b1_ref, w2_ref, b2_ref, o_ref):
    f32 = jnp.float32
    A = jnp.sum(a2_ref[:, :, :], axis=0)
    deg = jnp.sum(A, axis=1, keepdims=True) + 1.0
    dis = lax.rsqrt(deg)
    row = lax.broadcasted_iota(jnp.int32, (_N, _N), 0)
    col = lax.broadcasted_iota(jnp.int32, (_N, _N), 1)
    A = A[:, :_N] + jnp.where(row == col, 1.0, 0.0).astype(f32)
    z1 = dis * lax.dot_general(x_ref[:, :], w1_ref[:, :], _NT,
                               preferred_element_type=f32)
    h = jnp.maximum(
        dis * jnp.dot(A, z1, preferred_element_type=f32) + b1_ref[:, :], 0.0)
    z2 = dis * lax.dot_general(h, w2_ref[:, :], _NT,
                               preferred_element_type=f32)
    o_ref[:, :] = dis * jnp.dot(A, z2, preferred_element_type=f32) + b2_ref[:, :]


def kernel(x, edge_index, W1, b1, W2, b2):
    ei = edge_index.astype(jnp.int32)
    a_parts = _sc_count(ei[0], ei[1])
    out = pl.pallas_call(
        _tc_body,
        out_shape=jax.ShapeDtypeStruct((_N, W2.shape[0]), jnp.float32),
    )(a_parts.reshape(_NW, _N, _NCOL), x, W1, b1.reshape(1, -1),
      W2, b2.reshape(1, -1))
    return out.reshape(_N * W2.shape[0])


# speedup vs baseline: 6.9588x; 6.9588x over previous
import jax
import jax.numpy as jnp
from jax.experimental import pallas as pl

_N = 100
_E = 6400

_NT = (((1,), (1,)), ((), ()))


def _sg_kernel(x_ref, ei_ref, w1_ref, b1_ref, w2_ref, b2_ref, o_ref):
    f32 = jnp.float32
    src = ei_ref[0:1, :]
    dst = ei_ref[1:2, :]
    iota_ne = jax.lax.broadcasted_iota(jnp.int32, (_N, _E), 0)
    st = (src == iota_ne).astype(f32)
    dt = (dst == iota_ne).astype(f32)
    A = jax.lax.dot_general(dt, st, _NT, preferred_element_type=f32)
    row = jax.lax.broadcasted_iota(jnp.int32, (_N, _N), 0)
    col = jax.lax.broadcasted_iota(jnp.int32, (_N, _N), 1)
    A = A + jnp.where(row == col, 1.0, 0.0).astype(f32)
    deg = jnp.sum(A, axis=1, keepdims=True)
    dis = jax.lax.rsqrt(deg)
    z1 = dis * jax.lax.dot_general(x_ref[:, :], w1_ref[:, :], _NT,
                                   preferred_element_type=f32)
    h = jnp.maximum(
        dis * jnp.dot(A, z1, preferred_element_type=f32) + b1_ref[:, :], 0.0)
    z2 = dis * jax.lax.dot_general(h, w2_ref[:, :], _NT,
                                   preferred_element_type=f32)
    o_ref[:, :] = dis * jnp.dot(A, z2, preferred_element_type=f32) + b2_ref[:, :]


def kernel(x, edge_index, W1, b1, W2, b2):
    out = pl.pallas_call(
        _sg_kernel,
        out_shape=jax.ShapeDtypeStruct((_N, W2.shape[0]), jnp.float32),
    )(x, edge_index.astype(jnp.int32), W1, b1.reshape(1, -1),
      W2, b2.reshape(1, -1))
    return out.reshape(_N * W2.shape[0])
